# baseline (speedup 1.0000x reference)
import jax
import jax.numpy as jnp
from jax import lax
from jax.experimental import pallas as pl
from jax.experimental.pallas import tpu as pltpu
from jax.experimental.pallas import tpu_sc as plsc

NC = 2
NS = 16
N_ROWS = 100000
D = 128
NSEG = 512
DC = D // NC
BLK = 128
SB = 2 * BLK
NSUP = N_ROWS // SB
LEFT = 99840
TAIL = 32
SEG_PT = NSEG // NS
S = 5
IDSMAX = 25 * 2
OMAX = (NSUP // NS + S) // S + 1


def _body(x_hbm, b2_hbm, out_hbm, idx_v, tidx_v, buf_v, zero_v,
          sg0, sg1, sg2, sg3, sg4, ss0, ss1, ss2, ss3, ss4, shared):
    sem_g = (sg0, sg1, sg2, sg3, sg4)
    sem_s = (ss0, ss1, ss2, ss3, ss4)
    c = lax.axis_index("c")
    s = lax.axis_index("s")
    col0 = c * DC

    zeros = jnp.zeros((16,), jnp.float32)

    def zero_row(i, _):
        for j in range(DC // 16):
            zero_v[i, pl.ds(16 * j, 16)] = zeros
        return 0

    lax.fori_loop(0, SEG_PT, zero_row, 0)
    pltpu.sync_copy(zero_v, shared.at[pl.ds(s * SEG_PT, SEG_PT)])

    p0 = lax.div(NSUP * s, NS)
    p1 = lax.div(NSUP * (s + 1), NS)

    pltpu.sync_copy(b2_hbm.at[pl.ds(2 * p0, IDSMAX)], idx_v)
    plsc.subcore_barrier()

    def gather(p, si):
        row0 = pl.multiple_of(p * SB, 8)
        pltpu.async_copy(x_hbm.at[pl.ds(row0, SB), pl.ds(col0, DC)],
                         buf_v.at[si], sem_g[si])

    def wait_g(si):
        pltpu.make_async_copy(x_hbm.at[pl.ds(0, SB), pl.ds(0, DC)],
                              buf_v.at[si], sem_g[si]).wait()

    def scat(p, si):
        q = 2 * (p - p0)
        pltpu.async_copy(buf_v.at[si, pl.ds(0, BLK)],
                         shared.at[idx_v.at[q]], sem_s[si], add=True)
        pltpu.async_copy(buf_v.at[si, pl.ds(BLK, BLK)],
                         shared.at[idx_v.at[q + 1]], sem_s[si], add=True)

    def wait_s(si):
        pltpu.make_async_copy(x_hbm.at[pl.ds(0, SB), pl.ds(0, DC)],
                              buf_v.at[si], sem_s[si]).wait()

    gather(p0, 0)
    gather(p0 + 1, 1)
    gather(p0 + 2, 2)

    def outer(o, _):
        for si in range(S):
            p = p0 + S * o + si

            @pl.when(p < p1)
            def _():
                wait_g(si)
                scat(p, si)
                j = p + 3
                sj = (si + 3) % S

                @pl.when(j < p1)
                def _():
                    @pl.when(j - S >= p0)
                    def _():
                        wait_s(sj)

                    gather(j, sj)
        return 0

    lax.fori_loop(0, OMAX, outer, 0)

    for si in range(S):
        wait_s(si)

    @pl.when(s == NS - 1)
    def _():
        pltpu.sync_copy(x_hbm.at[pl.ds(LEFT, BLK), pl.ds(col0, DC)],
                        buf_v.at[0, pl.ds(0, BLK)])
        pltpu.sync_copy(b2_hbm.at[pl.ds(LEFT // BLK, 1)],
                        tidx_v.at[pl.ds(0, 1)])
        pltpu.sync_copy(buf_v.at[0, pl.ds(0, BLK)],
                        shared.at[tidx_v.at[0]], add=True)
        pltpu.sync_copy(x_hbm.at[pl.ds(LEFT + BLK, TAIL), pl.ds(col0, DC)],
                        buf_v.at[0, pl.ds(0, TAIL)])
        pltpu.sync_copy(b2_hbm.at[pl.ds(LEFT // BLK + 1, 1)],
                        tidx_v.at[pl.ds(1, 1)])
        pltpu.sync_copy(buf_v.at[0, pl.ds(0, TAIL)],
                        shared.at[tidx_v.at[1, pl.ds(0, TAIL)]], add=True)

    plsc.subcore_barrier()

    pltpu.sync_copy(shared.at[pl.ds(s * SEG_PT, SEG_PT)], zero_v)
    pltpu.sync_copy(zero_v,
                    out_hbm.at[pl.ds(s * SEG_PT, SEG_PT), pl.ds(col0, DC)])


@jax.jit
def _run(x, batch):
    mesh = plsc.VectorSubcoreMesh(core_axis_name="c", subcore_axis_name="s",
                                  num_cores=NC, num_subcores=NS)
    f = pl.kernel(
        _body,
        out_type=jax.ShapeDtypeStruct((NSEG, D), jnp.float32),
        mesh=mesh,
        compiler_params=pltpu.CompilerParams(use_tc_tiling_on_sc=False),
        scratch_types=[
            pltpu.VMEM((IDSMAX, BLK), jnp.int32),
            pltpu.VMEM((2, BLK), jnp.int32),
            pltpu.VMEM((S, SB, DC), jnp.float32),
            pltpu.VMEM((SEG_PT, DC), jnp.float32),
            pltpu.SemaphoreType.DMA,
            pltpu.SemaphoreType.DMA,
            pltpu.SemaphoreType.DMA,
            pltpu.SemaphoreType.DMA,
            pltpu.SemaphoreType.DMA,
            pltpu.SemaphoreType.DMA,
            pltpu.SemaphoreType.DMA,
            pltpu.SemaphoreType.DMA,
            pltpu.SemaphoreType.DMA,
            pltpu.SemaphoreType.DMA,
            pltpu.VMEM_SHARED((NSEG, DC), jnp.float32),
        ],
    )
    b2 = jnp.concatenate(
        [batch, jnp.zeros((BLK - TAIL,), jnp.int32)]).reshape(-1, BLK)
    return f(x, b2)


def kernel(x, batch):
    return _run(x, jnp.asarray(batch, jnp.int32))


# speedup vs baseline: 1.4470x; 1.4470x over previous
import jax
import jax.numpy as jnp
from jax import lax
from jax.experimental import pallas as pl
from jax.experimental.pallas import tpu as pltpu
from jax.experimental.pallas import tpu_sc as plsc

NC = 2
NS = 16
N_ROWS = 100000
D = 128
NSEG = 512
DC = D // NC
BLK = 128
SEG_PT = NSEG // NS
S = 6

NSC = 81920
NFULL = NSC // BLK
OMAX = (NFULL // NS + S) // S + 1

BR = 512
GRID_TC = (N_ROWS - NSC + BR - 1) // BR
PAD_TC = GRID_TC * BR
TC_OFF = NSC // BR


def _sc_body(x_hbm, b_hbm, out_hbm, idx_v, buf_v, zero_v,
             sg0, sg1, sg2, sg3, sg4, sg5, ss0, ss1, ss2, ss3, ss4, ss5,
             shared):
    sem_g = (sg0, sg1, sg2, sg3, sg4, sg5)
    sem_s = (ss0, ss1, ss2, ss3, ss4, ss5)
    c = lax.axis_index("c")
    s = lax.axis_index("s")
    col0 = c * DC

    zeros = jnp.zeros((16,), jnp.float32)

    def zero_row(i, _):
        for j in range(DC // 16):
            zero_v[i, pl.ds(16 * j, 16)] = zeros
        return 0

    lax.fori_loop(0, SEG_PT, zero_row, 0)
    pltpu.sync_copy(zero_v, shared.at[pl.ds(s * SEG_PT, SEG_PT)])
    plsc.subcore_barrier()

    b0 = lax.div(NFULL * s, NS)
    b1 = lax.div(NFULL * (s + 1), NS)

    def gather(k, si):
        row0 = pl.multiple_of(k * BLK, 8)
        pltpu.async_copy(x_hbm.at[pl.ds(row0, BLK), pl.ds(col0, DC)],
                         buf_v.at[si], sem_g[si])
        pltpu.async_copy(b_hbm.at[pl.ds(row0, BLK)], idx_v.at[si], sem_g[si])

    def wait_g(si):
        pltpu.make_async_copy(x_hbm.at[pl.ds(0, BLK), pl.ds(0, DC)],
                              buf_v.at[si], sem_g[si]).wait()
        pltpu.make_async_copy(b_hbm.at[pl.ds(0, BLK)],
                              idx_v.at[si], sem_g[si]).wait()

    def scat(si):
        pltpu.async_copy(buf_v.at[si], shared.at[idx_v.at[si]], sem_s[si],
                         add=True)

    def wait_s(si):
        pltpu.make_async_copy(x_hbm.at[pl.ds(0, BLK), pl.ds(0, DC)],
                              buf_v.at[si], sem_s[si]).wait()

    gather(b0, 0)
    gather(b0 + 1, 1)
    gather(b0 + 2, 2)
    gather(b0 + 3, 3)

    def outer(o, _):
        for si in range(S):
            k = b0 + S * o + si

            @pl.when(k < b1)
            def _():
                wait_g(si)
                scat(si)
                j = k + 4
                sj = (si + 4) % S

                @pl.when(j < b1)
                def _():
                    @pl.when(j - S >= b0)
                    def _():
                        wait_s(sj)

                    gather(j, sj)
        return 0

    lax.fori_loop(0, OMAX, outer, 0)

    for si in range(S):
        wait_s(si)

    plsc.subcore_barrier()

    pltpu.sync_copy(shared.at[pl.ds(s * SEG_PT, SEG_PT)], zero_v)
    pltpu.sync_copy(zero_v,
                    out_hbm.at[pl.ds(s * SEG_PT, SEG_PT), pl.ds(col0, DC)])


def _tc_body(ids_ref, x_ref, out_ref):
    i = pl.program_id(0)
    ids = ids_ref[0, 0, :]
    rows = (TC_OFF + i) * BR + lax.broadcasted_iota(jnp.int32, (BR, 1), 0)
    xb = jnp.where(rows < N_ROWS, x_ref[...], 0.0)
    segs = lax.broadcasted_iota(jnp.int32, (NSEG, BR), 0)
    oh = (segs == ids[None, :]).astype(jnp.bfloat16)
    part = lax.dot_general(oh, xb.astype(jnp.bfloat16),
                           (((1,), (0,)), ((), ())),
                           preferred_element_type=jnp.float32)

    @pl.when(i == 0)
    def _():
        out_ref[...] = part

    @pl.when(i > 0)
    def _():
        out_ref[...] += part


def _add_body(a_ref, b_ref, out_ref):
    out_ref[...] = a_ref[...] + b_ref[...]


@jax.jit
def _run(x, batch):
    mesh = plsc.VectorSubcoreMesh(core_axis_name="c", subcore_axis_name="s",
                                  num_cores=NC, num_subcores=NS)
    sc_part = pl.kernel(
        _sc_body,
        out_type=jax.ShapeDtypeStruct((NSEG, D), jnp.float32),
        mesh=mesh,
        compiler_params=pltpu.CompilerParams(use_tc_tiling_on_sc=False),
        scratch_types=[
            pltpu.VMEM((S, BLK), jnp.int32),
            pltpu.VMEM((S, BLK, DC), jnp.float32),
            pltpu.VMEM((SEG_PT, DC), jnp.float32),
            pltpu.SemaphoreType.DMA,
            pltpu.SemaphoreType.DMA,
            pltpu.SemaphoreType.DMA,
            pltpu.SemaphoreType.DMA,
            pltpu.SemaphoreType.DMA,
            pltpu.SemaphoreType.DMA,
            pltpu.SemaphoreType.DMA,
            pltpu.SemaphoreType.DMA,
            pltpu.SemaphoreType.DMA,
            pltpu.SemaphoreType.DMA,
            pltpu.SemaphoreType.DMA,
            pltpu.SemaphoreType.DMA,
            pltpu.VMEM_SHARED((NSEG, DC), jnp.float32),
        ],
    )(x, batch)

    ids3 = jnp.concatenate(
        [batch[NSC:],
         jnp.zeros((PAD_TC - (N_ROWS - NSC),), jnp.int32)]).reshape(
             GRID_TC, 1, BR)
    tc_part = pl.pallas_call(
        _tc_body,
        grid=(GRID_TC,),
        in_specs=[
            pl.BlockSpec((1, 1, BR), lambda i: (i, 0, 0)),
            pl.BlockSpec((BR, D), lambda i: (TC_OFF + i, 0)),
        ],
        out_specs=pl.BlockSpec((NSEG, D), lambda i: (0, 0)),
        out_shape=jax.ShapeDtypeStruct((NSEG, D), jnp.float32),
    )(ids3, x)

    return pl.pallas_call(
        _add_body,
        out_shape=jax.ShapeDtypeStruct((NSEG, D), jnp.float32),
    )(sc_part, tc_part)


def kernel(x, batch):
    return _run(x, jnp.asarray(batch, jnp.int32))
